# Initial kernel scaffold; baseline (speedup 1.0000x reference)
import jax
import jax.numpy as jnp
from jax.experimental import pallas as pl


def kernel(x, emb):
    raise NotImplementedError("write your pallas kernel here")


# speedup vs baseline: 5.0914x; 5.0914x over previous
import functools

import jax
import jax.numpy as jnp
from jax import lax
from jax.experimental import pallas as pl
from jax.experimental.pallas import tpu as pltpu
from jax.experimental.pallas import tpu_sc as plsc

S = 512
D = 256
MAX_LEN = 2048
NC = 2
NS = 16
NW = NC * NS
IPW = S // NW
C = 64
NCH = S // C
EWIN = C + IPW
L = 16


def _body(x_hbm, emb_hbm, out_hbm, x_v, emb_v, rows_v, sem):
    wid = lax.axis_index("s") * NC + lax.axis_index("c")
    i_base = wid * IPW

    for ch in range(NCH):
        j0 = ch * C
        pltpu.sync_copy(x_hbm.at[pl.ds(j0, C)], x_v)
        start = (MAX_LEN - 1) - (C - 1) + i_base - j0
        pltpu.sync_copy(emb_hbm.at[pl.ds(start, EWIN)], emb_v)

        def per_i(i_loc, _):
            b = i_loc & 1

            @pl.when(i_loc >= 2)
            def _wait_prev():
                pltpu.make_async_copy(
                    rows_v.at[b], out_hbm.at[pl.ds(0, C)], sem
                ).wait()

            @plsc.parallel_loop(0, C, 1, unroll=2)
            def per_jj(jj):
                r = (C - 1) + i_loc - jj
                for c in range(0, D, L):
                    rows_v[b, jj, pl.ds(c, L)] = (
                        emb_v[r, pl.ds(c, L)] + x_v[jj, pl.ds(c, L)]
                    )
            out_base = (i_base + i_loc) * S + j0
            pltpu.async_copy(rows_v.at[b], out_hbm.at[pl.ds(out_base, C)], sem)
            return 0

        lax.fori_loop(0, IPW, per_i, 0)

        for _ in range(2):
            pltpu.make_async_copy(
                rows_v.at[0], out_hbm.at[pl.ds(0, C)], sem
            ).wait()


def kernel(x, emb):
    x2 = x.reshape(S, D)
    mesh = plsc.VectorSubcoreMesh(core_axis_name="c", subcore_axis_name="s")
    run = functools.partial(
        pl.kernel,
        mesh=mesh,
        out_type=jax.ShapeDtypeStruct((S * S, D), jnp.float32),
        scratch_types=[
            pltpu.VMEM((C, D), jnp.float32),
            pltpu.VMEM((EWIN, D), jnp.float32),
            pltpu.VMEM((2, C, D), jnp.float32),
            pltpu.SemaphoreType.DMA,
        ],
    )(_body)
    out = run(x2, emb)
    return out.reshape(S, S, D)
